# Initial kernel scaffold; baseline (speedup 1.0000x reference)
import jax
import jax.numpy as jnp
from jax.experimental import pallas as pl


def kernel(x, x_classes, tables, W1, b1, W2, b2, W3, b3):
    raise NotImplementedError("write your pallas kernel here")


# speedup vs baseline: 2.0356x; 2.0356x over previous
import functools

import jax
import jax.numpy as jnp
from jax import lax
from jax.experimental import pallas as pl
from jax.experimental.pallas import tpu as pltpu
from jax.experimental.pallas import tpu_sc as plsc

_CH = 128
_G = 4


def _sc_gather(table_flat, idx2d, n_rows, emb_d):
    info = plsc.get_sparse_core_info()
    nc, ns = info.num_cores, info.num_subcores
    nw = nc * ns
    per_w = n_rows // nw
    k = per_w // _CH
    ng = k // _G
    assert per_w % _CH == 0 and k % _G == 0 and ng % 2 == 0

    mesh = plsc.VectorSubcoreMesh(core_axis_name="c", subcore_axis_name="s")

    @functools.partial(
        pl.kernel,
        out_type=jax.ShapeDtypeStruct((n_rows, emb_d), jnp.float32),
        mesh=mesh,
        compiler_params=pltpu.CompilerParams(use_tc_tiling_on_sc=False),
        scratch_types=[
            pltpu.VMEM((k, _CH), jnp.int32),
            pltpu.VMEM((_G * _CH, emb_d), jnp.float32),
            pltpu.VMEM((_G * _CH, emb_d), jnp.float32),
            pltpu.SemaphoreType.DMA,
            pltpu.SemaphoreType.DMA,
        ],
    )
    def gather_kernel(table_hbm, idx_hbm, out_hbm, idx_v, buf0, buf1, sem0, sem1):
        wid = lax.axis_index("s") * nc + lax.axis_index("c")
        row0 = wid * per_w
        pltpu.sync_copy(idx_hbm.at[pl.ds(wid * k, k)], idx_v)
        bufs = (buf0, buf1)
        sems = (sem0, sem1)

        def fire(g, buf, sem):
            for b in range(_G):
                pltpu.async_copy(
                    table_hbm.at[idx_v.at[g * _G + b]],
                    buf.at[pl.ds(b * _CH, _CH)],
                    sem)

        def drain(g, buf, sem):
            for b in range(_G):
                pltpu.make_async_copy(
                    table_hbm.at[idx_v.at[g * _G + b]],
                    buf.at[pl.ds(b * _CH, _CH)],
                    sem).wait()

        fire(0, buf0, sem0)
        fire(1, buf1, sem1)

        def outer(p, carry):
            for b in range(2):
                gg = p * 2 + b
                drain(gg, bufs[b], sems[b])
                pltpu.sync_copy(
                    bufs[b], out_hbm.at[pl.ds(row0 + gg * (_G * _CH), _G * _CH)])

                @pl.when(gg + 2 < ng)
                def _():
                    fire(gg + 2, bufs[b], sems[b])
            return carry

        lax.fori_loop(0, ng // 2, outer, 0)

    return gather_kernel(table_flat, idx2d)


def _mlp(emb, x, w1e, w1x, b1, w2, b2, w3, b3):
    bsz, d_emb = emb.shape
    bt = 2048
    h1 = w1e.shape[1]
    h2 = w2.shape[1]

    def body(emb_ref, x_ref, w1e_ref, w1x_ref, b1_ref, w2_ref, b2_ref,
             w3_ref, b3_ref, out_ref):
        h = jnp.dot(emb_ref[...], w1e_ref[...], preferred_element_type=jnp.float32)
        h = h + jnp.dot(x_ref[...], w1x_ref[...], preferred_element_type=jnp.float32)
        h = jnp.maximum(h + b1_ref[...], 0.0)
        h = jnp.maximum(
            jnp.dot(h, w2_ref[...], preferred_element_type=jnp.float32) + b2_ref[...],
            0.0)
        out_ref[...] = (
            jnp.dot(h, w3_ref[...], preferred_element_type=jnp.float32) + b3_ref[...])

    return pl.pallas_call(
        body,
        grid=(bsz // bt,),
        in_specs=[
            pl.BlockSpec((bt, d_emb), lambda i: (i, 0)),
            pl.BlockSpec((bt, x.shape[1]), lambda i: (i, 0)),
            pl.BlockSpec(w1e.shape, lambda i: (0, 0)),
            pl.BlockSpec(w1x.shape, lambda i: (0, 0)),
            pl.BlockSpec((1, h1), lambda i: (0, 0)),
            pl.BlockSpec(w2.shape, lambda i: (0, 0)),
            pl.BlockSpec((1, h2), lambda i: (0, 0)),
            pl.BlockSpec(w3.shape, lambda i: (0, 0)),
            pl.BlockSpec((1, 1), lambda i: (0, 0)),
        ],
        out_specs=pl.BlockSpec((bt, 1), lambda i: (i, 0)),
        out_shape=jax.ShapeDtypeStruct((bsz, 1), jnp.float32),
    )(emb, x, w1e, w1x, b1, w2, b2, w3, b3)


def kernel(x, x_classes, tables, W1, b1, W2, b2, W3, b3):
    bsz = x.shape[0]
    nf, vocab, emb_d = tables.shape
    table_flat = tables.reshape(nf * vocab, emb_d)
    idx_flat = (x_classes.astype(jnp.int32)
                + jnp.arange(nf, dtype=jnp.int32) * vocab).reshape(-1)
    idx2d = idx_flat.reshape(-1, _CH)
    rows = _sc_gather(table_flat, idx2d, bsz * nf, emb_d)
    emb = rows.reshape(bsz, nf * emb_d)
    w1e = W1[: nf * emb_d]
    w1x = W1[nf * emb_d:]
    return _mlp(emb, x, w1e, w1x, b1.reshape(1, -1), W2, b2.reshape(1, -1),
                W3, b3.reshape(1, 1))


# speedup vs baseline: 4.8429x; 2.3791x over previous
import functools

import jax
import jax.numpy as jnp
from jax import lax
from jax.experimental import pallas as pl
from jax.experimental.pallas import tpu as pltpu
from jax.experimental.pallas import tpu_sc as plsc

_LANES = 16
_QUART = 4096


def _sc_gather_t(t2, idx_flat, nf, vocab, emb_d, bsz):
    info = plsc.get_sparse_core_info()
    nc, ns = info.num_cores, info.num_subcores
    nw = nc * ns
    assert emb_d == nw
    nq = bsz // _QUART
    assert nq >= 2 and bsz % _QUART == 0
    mesh = plsc.VectorSubcoreMesh(core_axis_name="c", subcore_axis_name="s")

    @functools.partial(
        pl.kernel,
        out_type=jax.ShapeDtypeStruct((nf * emb_d * bsz,), jnp.float32),
        mesh=mesh,
        compiler_params=pltpu.CompilerParams(
            use_tc_tiling_on_sc=True, needs_layout_passes=False),
        scratch_types=[
            pltpu.VMEM((vocab,), jnp.float32),
            pltpu.VMEM((bsz,), jnp.int32),
            pltpu.VMEM((_QUART,), jnp.float32),
            pltpu.VMEM((_QUART,), jnp.float32),
            pltpu.SemaphoreType.DMA,
            pltpu.SemaphoreType.DMA,
            pltpu.SemaphoreType.DMA,
            pltpu.SemaphoreType.DMA,
        ],
    )
    def gather_kernel(t2_hbm, idx_hbm, out_hbm, row_v, idx_v, ob0, ob1,
                      rsem, isem, osem0, osem1):
        t = lax.axis_index("s") * nc + lax.axis_index("c")
        obs = (ob0, ob1)
        osems = (osem0, osem1)

        for i in range(nf):
            p = i * emb_d + t
            cp_i = pltpu.async_copy(
                idx_hbm.at[pl.ds(i * bsz, bsz)], idx_v, isem)
            cp_r = pltpu.async_copy(t2_hbm.at[p], row_v, rsem)
            cp_i.wait()
            cp_r.wait()
            for q in range(nq):
                ob, osem = obs[q % 2], osems[q % 2]
                dst = out_hbm.at[pl.ds(p * bsz + q * _QUART, _QUART)]
                if q >= 2 or i > 0:
                    pltpu.make_async_copy(ob, dst, osem).wait()

                def inner(j, carry):
                    b0 = j * _LANES
                    idx16 = idx_v[pl.ds(q * _QUART + b0, _LANES)]
                    ob[pl.ds(b0, _LANES)] = plsc.load_gather(row_v, [idx16])
                    return carry

                lax.fori_loop(0, _QUART // _LANES, inner, 0, unroll=8)
                pltpu.async_copy(ob, dst, osem)
        for b in range(2):
            q = nq - 2 + b
            p_last = (nf - 1) * emb_d + t
            pltpu.make_async_copy(
                obs[q % 2],
                out_hbm.at[pl.ds(p_last * bsz + q * _QUART, _QUART)],
                osems[q % 2]).wait()

    return gather_kernel(t2, idx_flat)


def _mlp_t(emb4, xt, w1et, w1xt, b1c, w2t, b2c, w3t, b3c):
    n_steps = emb4.shape[1]
    d_emb = emb4.shape[0]

    def body(emb_ref, xt_ref, w1et_ref, w1xt_ref, b1_ref, w2t_ref, b2_ref,
             w3t_ref, b3_ref, out_ref):
        e = emb_ref[...].reshape(d_emb, 128)
        h = jnp.dot(w1et_ref[...], e, preferred_element_type=jnp.float32)
        h = h + jnp.dot(w1xt_ref[...], xt_ref[...],
                        preferred_element_type=jnp.float32)
        h = jnp.maximum(h + b1_ref[...], 0.0)
        h = jnp.maximum(
            jnp.dot(w2t_ref[...], h, preferred_element_type=jnp.float32)
            + b2_ref[...], 0.0)
        out_ref[...] = (
            jnp.dot(w3t_ref[...], h, preferred_element_type=jnp.float32)
            + b3_ref[...]).reshape(1, 1, 128)

    return pl.pallas_call(
        body,
        grid=(n_steps,),
        in_specs=[
            pl.BlockSpec((d_emb, 1, 1, 128), lambda i: (0, i, 0, 0)),
            pl.BlockSpec((xt.shape[0], 128), lambda i: (0, i)),
            pl.BlockSpec(w1et.shape, lambda i: (0, 0)),
            pl.BlockSpec(w1xt.shape, lambda i: (0, 0)),
            pl.BlockSpec(b1c.shape, lambda i: (0, 0)),
            pl.BlockSpec(w2t.shape, lambda i: (0, 0)),
            pl.BlockSpec(b2c.shape, lambda i: (0, 0)),
            pl.BlockSpec(w3t.shape, lambda i: (0, 0)),
            pl.BlockSpec(b3c.shape, lambda i: (0, 0)),
        ],
        out_specs=pl.BlockSpec((1, 1, 128), lambda i: (i, 0, 0)),
        out_shape=jax.ShapeDtypeStruct((n_steps, 1, 128), jnp.float32),
    )(emb4, xt, w1et, w1xt, b1c, w2t, b2c, w3t, b3c)


def kernel(x, x_classes, tables, W1, b1, W2, b2, W3, b3):
    bsz = x.shape[0]
    nf, vocab, emb_d = tables.shape
    d_emb = nf * emb_d
    t2 = tables.transpose(0, 2, 1).reshape(d_emb, vocab)
    idx_flat = x_classes.T.reshape(-1)
    flat = _sc_gather_t(t2, idx_flat, nf, vocab, emb_d, bsz)
    emb4 = flat.reshape(d_emb, bsz // 128, 1, 128)
    xt = x.T
    w1et = W1[:d_emb].T
    w1xt = W1[d_emb:].T
    outt = _mlp_t(emb4, xt, w1et, w1xt, b1.reshape(-1, 1), W2.T,
                  b2.reshape(-1, 1), W3.T, b3.reshape(1, 1))
    return outt.reshape(bsz, 1)
